# baseline (speedup 1.0000x reference)
import functools

import jax
import jax.numpy as jnp
from jax import lax
from jax.experimental import pallas as pl
from jax.experimental.pallas import tpu as pltpu
from jax.experimental.pallas import tpu_sc as plsc

N_NODES = 10000
NODE_DIM = 128
N_EDGES = 320000
MAX_DEG = 512

NH = 10240
N_CORES = 2
N_SUB = 16
NW = N_CORES * N_SUB
EPT = 9984
EPT_LAST = N_EDGES - (NW - 1) * EPT
SLICE = NH // N_SUB


def _hist_body(edges_hbm, out_hbm,
               buf_v, priv_in, priv_out, acc_in, acc_out,
               stage_in_sh, stage_out_sh, sem):
    c = lax.axis_index("c")
    s = lax.axis_index("s")
    g = c * N_SUB + s

    cp = pltpu.make_async_copy(
        edges_hbm.at[:, pl.ds(g * EPT, EPT_LAST)], buf_v, sem)
    cp.start()

    zeros = jnp.zeros((16,), jnp.int32)

    @plsc.parallel_loop(0, NH // 16, unroll=8)
    def zloop(i):
        priv_in[pl.ds(i * 16, 16)] = zeros
        priv_out[pl.ds(i * 16, 16)] = zeros

    cp.wait()

    n_vec = jnp.where(g == NW - 1, EPT_LAST // 16, EPT // 16)

    @plsc.parallel_loop(0, n_vec, unroll=8)
    def hloop(i):
        ids_i = buf_v[1, pl.ds(i * 16, 16)]
        cnt_i, last_i = plsc.scan_count(ids_i)
        plsc.addupdate_scatter(priv_in, [ids_i], cnt_i, mask=last_i)
        ids_o = buf_v[0, pl.ds(i * 16, 16)]
        cnt_o, last_o = plsc.scan_count(ids_o)
        plsc.addupdate_scatter(priv_out, [ids_o], cnt_o, mask=last_o)

    p1 = pltpu.make_async_copy(priv_in, stage_in_sh.at[s], sem)
    p2 = pltpu.make_async_copy(priv_out, stage_out_sh.at[s], sem)
    p1.start()
    p2.start()
    p1.wait()
    p2.wait()
    plsc.subcore_barrier()

    copies = []
    for t in range(N_SUB):
        c1 = pltpu.make_async_copy(stage_in_sh.at[t, pl.ds(s * SLICE, SLICE)],
                                   buf_v.at[1, pl.ds(t * SLICE, SLICE)], sem)
        c2 = pltpu.make_async_copy(stage_out_sh.at[t, pl.ds(s * SLICE, SLICE)],
                                   buf_v.at[0, pl.ds(t * SLICE, SLICE)], sem)
        c1.start()
        c2.start()
        copies.append(c1)
        copies.append(c2)
    for cpi in copies:
        cpi.wait()

    @plsc.parallel_loop(0, SLICE // 16, unroll=2)
    def rloop(k):
        base = k * 16
        a = buf_v[1, pl.ds(base, 16)]
        b = buf_v[0, pl.ds(base, 16)]
        for t in range(1, N_SUB):
            a = a + buf_v[1, pl.ds(t * SLICE + base, 16)]
            b = b + buf_v[0, pl.ds(t * SLICE + base, 16)]
        acc_in[pl.ds(base, 16)] = a
        acc_out[pl.ds(base, 16)] = b

    w1 = pltpu.make_async_copy(acc_in, out_hbm.at[c, pl.ds(s * SLICE, SLICE)], sem)
    w2 = pltpu.make_async_copy(acc_out, out_hbm.at[2 + c, pl.ds(s * SLICE, SLICE)], sem)
    w1.start()
    w2.start()
    w1.wait()
    w2.wait()


_hist = functools.partial(
    pl.kernel,
    out_type=jax.ShapeDtypeStruct((4, NH), jnp.int32),
    mesh=plsc.VectorSubcoreMesh(core_axis_name="c", subcore_axis_name="s"),
    compiler_params=pltpu.CompilerParams(
        needs_layout_passes=False, use_tc_tiling_on_sc=True),
    scratch_types=[
        pltpu.VMEM((2, EPT_LAST), jnp.int32),
        pltpu.VMEM((NH,), jnp.int32),
        pltpu.VMEM((NH,), jnp.int32),
        pltpu.VMEM((SLICE,), jnp.int32),
        pltpu.VMEM((SLICE,), jnp.int32),
        pltpu.VMEM_SHARED((N_SUB, NH), jnp.int32),
        pltpu.VMEM_SHARED((N_SUB, NH), jnp.int32),
        pltpu.SemaphoreType.DMA,
    ],
)(_hist_body)


BN = 10240
DC = 128


def _enc_body(h_ref, x_ref, zin_ref, zout_ref, o_ref):
    h = h_ref[...]
    deg_in = jnp.minimum(h[0] + h[1], MAX_DEG - 1)
    deg_out = jnp.minimum(h[2] + h[3], MAX_DEG - 1)
    dm_in = jnp.max(deg_in)
    dm_out = jnp.max(deg_out)
    iota = lax.broadcasted_iota(jnp.int32, (BN, DC), 1)
    o_ref[...] = x_ref[...]

    for k in range(MAX_DEG // DC):
        @pl.when(dm_in >= k * DC)
        def _():
            oh = (deg_in[:, None] == iota + (k * DC)).astype(jnp.bfloat16)
            zb = zin_ref[pl.ds(k * DC, DC), :].astype(jnp.bfloat16)
            o_ref[...] += jnp.dot(oh, zb, preferred_element_type=jnp.float32)

        @pl.when(dm_out >= k * DC)
        def _():
            oh = (deg_out[:, None] == iota + (k * DC)).astype(jnp.bfloat16)
            zb = zout_ref[pl.ds(k * DC, DC), :].astype(jnp.bfloat16)
            o_ref[...] += jnp.dot(oh, zb, preferred_element_type=jnp.float32)


def kernel(x, edge_index, z_in, z_out):
    e = edge_index.astype(jnp.int32)
    h4 = _hist(e)

    return pl.pallas_call(
        _enc_body,
        grid=(pl.cdiv(N_NODES, BN),),
        in_specs=[
            pl.BlockSpec((4, BN), lambda i: (0, i)),
            pl.BlockSpec((BN, NODE_DIM), lambda i: (i, 0)),
            pl.BlockSpec((MAX_DEG, NODE_DIM), lambda i: (0, 0)),
            pl.BlockSpec((MAX_DEG, NODE_DIM), lambda i: (0, 0)),
        ],
        out_specs=pl.BlockSpec((BN, NODE_DIM), lambda i: (i, 0)),
        out_shape=jax.ShapeDtypeStruct((N_NODES, NODE_DIM), jnp.float32),
    )(h4, x, z_in, z_out)


# speedup vs baseline: 1.0343x; 1.0343x over previous
import functools

import jax
import jax.numpy as jnp
from jax import lax
from jax.experimental import pallas as pl
from jax.experimental.pallas import tpu as pltpu
from jax.experimental.pallas import tpu_sc as plsc

N_NODES = 10000
NODE_DIM = 128
N_EDGES = 320000
MAX_DEG = 512

NH = 10240
N_CORES = 2
N_SUB = 16
NW = N_CORES * N_SUB
EPT = 9984
EPT_LAST = N_EDGES - (NW - 1) * EPT
SLICE = NH // N_SUB


def _hist_body(edges_hbm, out_hbm,
               buf_v, priv_in, priv_out, acc_in, acc_out,
               stage_in_sh, stage_out_sh, sem):
    c = lax.axis_index("c")
    s = lax.axis_index("s")
    g = c * N_SUB + s

    cp = pltpu.make_async_copy(
        edges_hbm.at[:, pl.ds(g * EPT, EPT_LAST)], buf_v, sem)
    cp.start()

    zeros = jnp.zeros((16,), jnp.int32)

    @plsc.parallel_loop(0, NH // 16, unroll=8)
    def zloop(i):
        priv_in[pl.ds(i * 16, 16)] = zeros
        priv_out[pl.ds(i * 16, 16)] = zeros

    cp.wait()

    n_vec = jnp.where(g == NW - 1, EPT_LAST // 16, EPT // 16)

    @plsc.parallel_loop(0, n_vec, unroll=8)
    def hloop_in(i):
        ids_i = buf_v[1, pl.ds(i * 16, 16)]
        cnt_i, last_i = plsc.scan_count(ids_i)
        plsc.addupdate_scatter(priv_in, [ids_i], cnt_i, mask=last_i)

    p1 = pltpu.make_async_copy(priv_in, stage_in_sh.at[s], sem)
    p1.start()

    @plsc.parallel_loop(0, n_vec, unroll=8)
    def hloop_out(i):
        ids_o = buf_v[0, pl.ds(i * 16, 16)]
        cnt_o, last_o = plsc.scan_count(ids_o)
        plsc.addupdate_scatter(priv_out, [ids_o], cnt_o, mask=last_o)

    p2 = pltpu.make_async_copy(priv_out, stage_out_sh.at[s], sem)
    p2.start()
    p1.wait()
    p2.wait()
    plsc.subcore_barrier()

    copies = []
    for t in range(N_SUB):
        c1 = pltpu.make_async_copy(stage_in_sh.at[t, pl.ds(s * SLICE, SLICE)],
                                   buf_v.at[1, pl.ds(t * SLICE, SLICE)], sem)
        c2 = pltpu.make_async_copy(stage_out_sh.at[t, pl.ds(s * SLICE, SLICE)],
                                   buf_v.at[0, pl.ds(t * SLICE, SLICE)], sem)
        c1.start()
        c2.start()
        copies.append(c1)
        copies.append(c2)
    for cpi in copies:
        cpi.wait()

    @plsc.parallel_loop(0, SLICE // 16, unroll=2)
    def rloop(k):
        base = k * 16
        a = buf_v[1, pl.ds(base, 16)]
        b = buf_v[0, pl.ds(base, 16)]
        for t in range(1, N_SUB):
            a = a + buf_v[1, pl.ds(t * SLICE + base, 16)]
            b = b + buf_v[0, pl.ds(t * SLICE + base, 16)]
        acc_in[pl.ds(base, 16)] = a
        acc_out[pl.ds(base, 16)] = b

    w1 = pltpu.make_async_copy(acc_in, out_hbm.at[c, pl.ds(s * SLICE, SLICE)], sem)
    w2 = pltpu.make_async_copy(acc_out, out_hbm.at[2 + c, pl.ds(s * SLICE, SLICE)], sem)
    w1.start()
    w2.start()
    w1.wait()
    w2.wait()


_hist = functools.partial(
    pl.kernel,
    out_type=jax.ShapeDtypeStruct((4, NH), jnp.int32),
    mesh=plsc.VectorSubcoreMesh(core_axis_name="c", subcore_axis_name="s"),
    compiler_params=pltpu.CompilerParams(
        needs_layout_passes=False, use_tc_tiling_on_sc=True),
    scratch_types=[
        pltpu.VMEM((2, EPT_LAST), jnp.int32),
        pltpu.VMEM((NH,), jnp.int32),
        pltpu.VMEM((NH,), jnp.int32),
        pltpu.VMEM((SLICE,), jnp.int32),
        pltpu.VMEM((SLICE,), jnp.int32),
        pltpu.VMEM_SHARED((N_SUB, NH), jnp.int32),
        pltpu.VMEM_SHARED((N_SUB, NH), jnp.int32),
        pltpu.SemaphoreType.DMA,
    ],
)(_hist_body)


BN = 5120
DC = 128


def _enc_body(h_ref, x_ref, zin_ref, zout_ref, o_ref):
    h = h_ref[...]
    deg_in = jnp.minimum(h[0] + h[1], MAX_DEG - 1)
    deg_out = jnp.minimum(h[2] + h[3], MAX_DEG - 1)
    dm_in = jnp.max(deg_in)
    dm_out = jnp.max(deg_out)
    iota = lax.broadcasted_iota(jnp.int32, (BN, DC), 1)
    o_ref[...] = x_ref[...]

    for k in range(MAX_DEG // DC):
        @pl.when(dm_in >= k * DC)
        def _():
            oh = (deg_in[:, None] == iota + (k * DC)).astype(jnp.bfloat16)
            zb = zin_ref[pl.ds(k * DC, DC), :].astype(jnp.bfloat16)
            o_ref[...] += jnp.dot(oh, zb, preferred_element_type=jnp.float32)

        @pl.when(dm_out >= k * DC)
        def _():
            oh = (deg_out[:, None] == iota + (k * DC)).astype(jnp.bfloat16)
            zb = zout_ref[pl.ds(k * DC, DC), :].astype(jnp.bfloat16)
            o_ref[...] += jnp.dot(oh, zb, preferred_element_type=jnp.float32)


def kernel(x, edge_index, z_in, z_out):
    e = edge_index.astype(jnp.int32)
    h4 = _hist(e)

    return pl.pallas_call(
        _enc_body,
        grid=(pl.cdiv(N_NODES, BN),),
        in_specs=[
            pl.BlockSpec((4, BN), lambda i: (0, i)),
            pl.BlockSpec((BN, NODE_DIM), lambda i: (i, 0)),
            pl.BlockSpec((MAX_DEG, NODE_DIM), lambda i: (0, 0)),
            pl.BlockSpec((MAX_DEG, NODE_DIM), lambda i: (0, 0)),
        ],
        out_specs=pl.BlockSpec((BN, NODE_DIM), lambda i: (i, 0)),
        out_shape=jax.ShapeDtypeStruct((N_NODES, NODE_DIM), jnp.float32),
    )(h4, x, z_in, z_out)


# speedup vs baseline: 1.0803x; 1.0444x over previous
import functools

import jax
import jax.numpy as jnp
from jax import lax
from jax.experimental import pallas as pl
from jax.experimental.pallas import tpu as pltpu
from jax.experimental.pallas import tpu_sc as plsc

N_NODES = 10000
NODE_DIM = 128
N_EDGES = 320000
MAX_DEG = 512

NH = 10240
N_CORES = 2
N_SUB = 16
NW = N_CORES * N_SUB
EPT = 9984
EPT_LAST = N_EDGES - (NW - 1) * EPT
SLICE = NH // N_SUB


def _hist_body(edges_hbm, out_hbm,
               buf_v, gbuf_v, priv_in, priv_out, acc_in, acc_out,
               stage_in_sh, stage_out_sh, sem):
    c = lax.axis_index("c")
    s = lax.axis_index("s")
    g = c * N_SUB + s

    cp = pltpu.make_async_copy(
        edges_hbm.at[:, pl.ds(g * EPT, EPT_LAST)], buf_v, sem)
    cp.start()

    zeros = jnp.zeros((16,), jnp.int32)

    @plsc.parallel_loop(0, NH // 16, unroll=8)
    def zloop(i):
        priv_in[pl.ds(i * 16, 16)] = zeros
        priv_out[pl.ds(i * 16, 16)] = zeros

    cp.wait()

    n_vec = jnp.where(g == NW - 1, EPT_LAST // 16, EPT // 16)

    @plsc.parallel_loop(0, n_vec, unroll=8)
    def hloop_in(i):
        ids_i = buf_v[1, pl.ds(i * 16, 16)]
        cnt_i, last_i = plsc.scan_count(ids_i)
        plsc.addupdate_scatter(priv_in, [ids_i], cnt_i, mask=last_i)

    p1 = pltpu.make_async_copy(priv_in, stage_in_sh.at[s], sem)
    p1.start()

    @plsc.parallel_loop(0, n_vec, unroll=8)
    def hloop_out(i):
        ids_o = buf_v[0, pl.ds(i * 16, 16)]
        cnt_o, last_o = plsc.scan_count(ids_o)
        plsc.addupdate_scatter(priv_out, [ids_o], cnt_o, mask=last_o)

    p2 = pltpu.make_async_copy(priv_out, stage_out_sh.at[s], sem)
    p2.start()
    p1.wait()
    p2.wait()
    plsc.subcore_barrier()

    g1 = pltpu.make_async_copy(stage_in_sh.at[:, pl.ds(s * SLICE, SLICE)],
                               gbuf_v.at[0], sem)
    g2 = pltpu.make_async_copy(stage_out_sh.at[:, pl.ds(s * SLICE, SLICE)],
                               gbuf_v.at[1], sem)
    g1.start()
    g2.start()
    g1.wait()
    g2.wait()

    @plsc.parallel_loop(0, SLICE // 16, unroll=2)
    def rloop(k):
        base = k * 16
        a = gbuf_v[0, 0, pl.ds(base, 16)]
        b = gbuf_v[1, 0, pl.ds(base, 16)]
        for t in range(1, N_SUB):
            a = a + gbuf_v[0, t, pl.ds(base, 16)]
            b = b + gbuf_v[1, t, pl.ds(base, 16)]
        acc_in[pl.ds(base, 16)] = a
        acc_out[pl.ds(base, 16)] = b

    w1 = pltpu.make_async_copy(acc_in, out_hbm.at[c, pl.ds(s * SLICE, SLICE)], sem)
    w2 = pltpu.make_async_copy(acc_out, out_hbm.at[2 + c, pl.ds(s * SLICE, SLICE)], sem)
    w1.start()
    w2.start()
    w1.wait()
    w2.wait()


_hist = functools.partial(
    pl.kernel,
    out_type=jax.ShapeDtypeStruct((4, NH), jnp.int32),
    mesh=plsc.VectorSubcoreMesh(core_axis_name="c", subcore_axis_name="s"),
    compiler_params=pltpu.CompilerParams(
        needs_layout_passes=False, use_tc_tiling_on_sc=True),
    scratch_types=[
        pltpu.VMEM((2, EPT_LAST), jnp.int32),
        pltpu.VMEM((2, N_SUB, SLICE), jnp.int32),
        pltpu.VMEM((NH,), jnp.int32),
        pltpu.VMEM((NH,), jnp.int32),
        pltpu.VMEM((SLICE,), jnp.int32),
        pltpu.VMEM((SLICE,), jnp.int32),
        pltpu.VMEM_SHARED((N_SUB, NH), jnp.int32),
        pltpu.VMEM_SHARED((N_SUB, NH), jnp.int32),
        pltpu.SemaphoreType.DMA,
    ],
)(_hist_body)


BN = 5120
DC = 128


def _enc_body(h_ref, x_ref, zin_ref, zout_ref, o_ref):
    h = h_ref[...]
    deg_in = jnp.minimum(h[0] + h[1], MAX_DEG - 1)
    deg_out = jnp.minimum(h[2] + h[3], MAX_DEG - 1)
    dm_in = jnp.max(deg_in)
    dm_out = jnp.max(deg_out)
    iota = lax.broadcasted_iota(jnp.int32, (BN, DC), 1)
    o_ref[...] = x_ref[...]

    for k in range(MAX_DEG // DC):
        @pl.when(dm_in >= k * DC)
        def _():
            oh = (deg_in[:, None] == iota + (k * DC)).astype(jnp.bfloat16)
            zb = zin_ref[pl.ds(k * DC, DC), :].astype(jnp.bfloat16)
            o_ref[...] += jnp.dot(oh, zb, preferred_element_type=jnp.float32)

        @pl.when(dm_out >= k * DC)
        def _():
            oh = (deg_out[:, None] == iota + (k * DC)).astype(jnp.bfloat16)
            zb = zout_ref[pl.ds(k * DC, DC), :].astype(jnp.bfloat16)
            o_ref[...] += jnp.dot(oh, zb, preferred_element_type=jnp.float32)


def kernel(x, edge_index, z_in, z_out):
    e = edge_index.astype(jnp.int32)
    h4 = _hist(e)

    return pl.pallas_call(
        _enc_body,
        grid=(pl.cdiv(N_NODES, BN),),
        in_specs=[
            pl.BlockSpec((4, BN), lambda i: (0, i)),
            pl.BlockSpec((BN, NODE_DIM), lambda i: (i, 0)),
            pl.BlockSpec((MAX_DEG, NODE_DIM), lambda i: (0, 0)),
            pl.BlockSpec((MAX_DEG, NODE_DIM), lambda i: (0, 0)),
        ],
        out_specs=pl.BlockSpec((BN, NODE_DIM), lambda i: (i, 0)),
        out_shape=jax.ShapeDtypeStruct((N_NODES, NODE_DIM), jnp.float32),
    )(h4, x, z_in, z_out)


# speedup vs baseline: 1.0842x; 1.0036x over previous
import functools

import jax
import jax.numpy as jnp
from jax import lax
from jax.experimental import pallas as pl
from jax.experimental.pallas import tpu as pltpu
from jax.experimental.pallas import tpu_sc as plsc

N_NODES = 10000
NODE_DIM = 128
N_EDGES = 320000
MAX_DEG = 512

NH = 10240
N_CORES = 2
N_SUB = 16
NW = N_CORES * N_SUB
EPT = 9984
EPT_LAST = N_EDGES - (NW - 1) * EPT
SLICE = NH // N_SUB


def _hist_body(edges_hbm, out_hbm,
               buf_v, gbuf_v, priv_in, priv_out, acc_in, acc_out,
               stage_in_sh, stage_out_sh, sem):
    c = lax.axis_index("c")
    s = lax.axis_index("s")
    g = c * N_SUB + s

    cp = pltpu.make_async_copy(
        edges_hbm.at[:, pl.ds(g * EPT, EPT_LAST)], buf_v, sem)
    cp.start()

    zeros = jnp.zeros((16,), jnp.int32)

    @plsc.parallel_loop(0, NH // 16, unroll=8)
    def zloop(i):
        priv_in[pl.ds(i * 16, 16)] = zeros
        priv_out[pl.ds(i * 16, 16)] = zeros

    cp.wait()

    n_vec = jnp.where(g == NW - 1, EPT_LAST // 16, EPT // 16)

    @plsc.parallel_loop(0, n_vec, unroll=8)
    def hloop_in(i):
        ids_i = buf_v[1, pl.ds(i * 16, 16)]
        cnt_i, last_i = plsc.scan_count(ids_i)
        plsc.addupdate_scatter(priv_in, [ids_i], cnt_i, mask=last_i)

    p1 = pltpu.make_async_copy(priv_in, stage_in_sh.at[s], sem)
    p1.start()

    @plsc.parallel_loop(0, n_vec, unroll=8)
    def hloop_out(i):
        ids_o = buf_v[0, pl.ds(i * 16, 16)]
        cnt_o, last_o = plsc.scan_count(ids_o)
        plsc.addupdate_scatter(priv_out, [ids_o], cnt_o, mask=last_o)

    p2 = pltpu.make_async_copy(priv_out, stage_out_sh.at[s], sem)
    p2.start()
    p1.wait()
    p2.wait()
    plsc.subcore_barrier()

    g1 = pltpu.make_async_copy(stage_in_sh.at[:, pl.ds(s * SLICE, SLICE)],
                               gbuf_v.at[0], sem)
    g2 = pltpu.make_async_copy(stage_out_sh.at[:, pl.ds(s * SLICE, SLICE)],
                               gbuf_v.at[1], sem)
    g1.start()
    g2.start()
    g1.wait()
    g2.wait()

    @plsc.parallel_loop(0, SLICE // 16, unroll=2)
    def rloop(k):
        base = k * 16
        a = gbuf_v[0, 0, pl.ds(base, 16)]
        b = gbuf_v[1, 0, pl.ds(base, 16)]
        for t in range(1, N_SUB):
            a = a + gbuf_v[0, t, pl.ds(base, 16)]
            b = b + gbuf_v[1, t, pl.ds(base, 16)]
        acc_in[pl.ds(base, 16)] = a
        acc_out[pl.ds(base, 16)] = b

    w1 = pltpu.make_async_copy(acc_in, out_hbm.at[c, pl.ds(s * SLICE, SLICE)], sem)
    w2 = pltpu.make_async_copy(acc_out, out_hbm.at[2 + c, pl.ds(s * SLICE, SLICE)], sem)
    w1.start()
    w2.start()
    w1.wait()
    w2.wait()


_hist = functools.partial(
    pl.kernel,
    out_type=jax.ShapeDtypeStruct((4, NH), jnp.int32),
    mesh=plsc.VectorSubcoreMesh(core_axis_name="c", subcore_axis_name="s"),
    compiler_params=pltpu.CompilerParams(
        needs_layout_passes=False, use_tc_tiling_on_sc=True),
    scratch_types=[
        pltpu.VMEM((2, EPT_LAST), jnp.int32),
        pltpu.VMEM((2, N_SUB, SLICE), jnp.int32),
        pltpu.VMEM((NH,), jnp.int32),
        pltpu.VMEM((NH,), jnp.int32),
        pltpu.VMEM((SLICE,), jnp.int32),
        pltpu.VMEM((SLICE,), jnp.int32),
        pltpu.VMEM_SHARED((N_SUB, NH), jnp.int32),
        pltpu.VMEM_SHARED((N_SUB, NH), jnp.int32),
        pltpu.SemaphoreType.DMA,
    ],
)(_hist_body)


BN = 5120
DC = 64


def _enc_body(h_ref, x_ref, zin_ref, zout_ref, o_ref):
    h = h_ref[...]
    deg_in = jnp.minimum(h[0] + h[1], MAX_DEG - 1)
    deg_out = jnp.minimum(h[2] + h[3], MAX_DEG - 1)
    dm_in = jnp.max(deg_in)
    dm_out = jnp.max(deg_out)
    iota = lax.broadcasted_iota(jnp.int32, (BN, DC), 1)
    o_ref[...] = x_ref[...]

    for k in range(MAX_DEG // DC):
        @pl.when(dm_in >= k * DC)
        def _():
            oh = (deg_in[:, None] == iota + (k * DC)).astype(jnp.bfloat16)
            zb = zin_ref[pl.ds(k * DC, DC), :].astype(jnp.bfloat16)
            o_ref[...] += jnp.dot(oh, zb, preferred_element_type=jnp.float32)

        @pl.when(dm_out >= k * DC)
        def _():
            oh = (deg_out[:, None] == iota + (k * DC)).astype(jnp.bfloat16)
            zb = zout_ref[pl.ds(k * DC, DC), :].astype(jnp.bfloat16)
            o_ref[...] += jnp.dot(oh, zb, preferred_element_type=jnp.float32)


def kernel(x, edge_index, z_in, z_out):
    e = edge_index.astype(jnp.int32)
    h4 = _hist(e)

    return pl.pallas_call(
        _enc_body,
        grid=(pl.cdiv(N_NODES, BN),),
        in_specs=[
            pl.BlockSpec((4, BN), lambda i: (0, i)),
            pl.BlockSpec((BN, NODE_DIM), lambda i: (i, 0)),
            pl.BlockSpec((MAX_DEG, NODE_DIM), lambda i: (0, 0)),
            pl.BlockSpec((MAX_DEG, NODE_DIM), lambda i: (0, 0)),
        ],
        out_specs=pl.BlockSpec((BN, NODE_DIM), lambda i: (i, 0)),
        out_shape=jax.ShapeDtypeStruct((N_NODES, NODE_DIM), jnp.float32),
    )(h4, x, z_in, z_out)
